# baseline (speedup 1.0000x reference)
import functools

import jax
import jax.numpy as jnp
from jax import lax
from jax.experimental import pallas as pl
from jax.experimental.pallas import tpu as pltpu
from jax.experimental.pallas import tpu_sc as plsc

D = 64
ROWS = 4096
COLS = 200
B = ROWS * COLS
NC = 2
NS = 16
NW = NC * NS
BPW = B // NW
SUB = 128
CHUNK = 512
NSUB = CHUNK // SUB
NG = BPW // CHUNK
SCALE = 8.0


def _sc_embed(x2d, table):
    mesh = plsc.VectorSubcoreMesh(core_axis_name="c", subcore_axis_name="s")

    @functools.partial(
        pl.kernel,
        mesh=mesh,
        compiler_params=pltpu.CompilerParams(use_tc_tiling_on_sc=False),
        out_type=jax.ShapeDtypeStruct((B, D), jnp.float32),
        scratch_types=[
            pltpu.VMEM((NSUB, SUB), jnp.int32),
            pltpu.VMEM((CHUNK, D), jnp.float32),
            pltpu.SemaphoreType.DMA,
        ],
    )
    def k(x_hbm, tbl_hbm, out_hbm, idx_v, rows_v, gsem):
        wid = lax.axis_index("s") * NC + lax.axis_index("c")
        base = wid * BPW
        xbase = wid * (BPW // SUB)

        def chunk_body(g, carry):
            row0 = base + g * CHUNK
            pltpu.sync_copy(x_hbm.at[pl.ds(xbase + g * NSUB, NSUB)], idx_v)
            copies = [
                pltpu.async_copy(
                    tbl_hbm.at[idx_v.at[j]],
                    rows_v.at[pl.ds(j * SUB, SUB)],
                    gsem,
                )
                for j in range(NSUB)
            ]
            for cp in copies:
                cp.wait()

            for j in range(NSUB):
                def grp_body(r8, c, j=j):
                    iv = idx_v[j, pl.ds(r8 * 16, 16)]
                    fv = jnp.where(iv == 0, jnp.float32(0.0),
                                   jnp.float32(SCALE))
                    for t in range(16):
                        f = fv[t]
                        row = j * SUB + r8 * 16 + t
                        for q in range(D // 16):
                            sl = pl.ds(q * 16, 16)
                            rows_v[row, sl] = rows_v[row, sl] * f
                    return c

                lax.fori_loop(0, SUB // 16, grp_body, 0)
            pltpu.sync_copy(rows_v, out_hbm.at[pl.ds(row0, CHUNK)])
            return carry

        lax.fori_loop(0, NG, chunk_body, 0)

    return k(x2d, table)


def kernel(x, table):
    x2d = x.reshape(B // SUB, SUB)
    out = _sc_embed(x2d, table)
    return out.reshape(ROWS, COLS, D)
